# baseline (speedup 1.0000x reference)
import numpy as np

import jax
import jax.numpy as jnp
from jax.experimental import pallas as pl
from jax.experimental.pallas import tpu as pltpu

B, N, T, C_IN = 8, 1000, 12, 64
IB = 5
IR = N // IB
OUT_CH = 4
ATT_DIM = T - 2
HALF = ATT_DIM * OUT_CH
LANE = 128

_SEL = np.zeros((3, T, ATT_DIM), dtype=np.float32)
for _k in range(3):
    for _t in range(ATT_DIM):
        _SEL[_k, _t + _k, _t] = 1.0


def _fused_kernel(x_ref, m_ref, brow_ref, wab_ref, wabt_ref, bfc_ref,
                  a_ref, out_ref, s1_scr, s2_scr):
    i = pl.program_id(1)

    @pl.when(i == 0)
    def _score():
        x = x_ref[0]
        y = jnp.dot(x, m_ref[...], preferred_element_type=jnp.float32)
        o1 = y[:, 0:LANE] + brow_ref[0, :][None, :]
        o2 = y[:, LANE:2 * LANE] + brow_ref[1, :][None, :]
        o3 = y[:, 2 * LANE:3 * LANE] + brow_ref[2, :][None, :]
        t = jax.nn.relu(o1 + jax.nn.sigmoid(o2) + o3)
        s1_scr[...] = jnp.dot(t, wabt_ref[...],
                              preferred_element_type=jnp.float32)[:, 0:1]
        s2_scr[...] = jax.lax.dot_general(
            wab_ref[...], t, (((1,), (1,)), ((), ())),
            preferred_element_type=jnp.float32)[1:2, :]

    s1c = s1_scr[pl.ds(i * IR, IR), :]
    raw = s1c + s2_scr[...] + bfc_ref[0, 0]
    scores = jnp.where(raw >= 0, raw, 0.01 * raw)
    a = a_ref[pl.ds(i * IR, IR), :]
    val = jnp.where(a != 0, scores, 0.0)
    m = jnp.max(val, axis=1, keepdims=True)
    e = jnp.exp(val - m)
    s = jnp.sum(e, axis=1, keepdims=True)
    out_ref[0] = e * (1.0 / s) * a


@jax.jit
def kernel(X, A, W1, b1, W2, b2, W3, b3, Wfc, bfc):
    W = jnp.stack([jnp.transpose(w[:, :, 0, :], (2, 1, 0))
                   for w in (W1, W2, W3)])
    sel = jnp.asarray(_SEL)
    M0 = jnp.einsum('kpt,jkco->pcjto', sel, W)
    M1 = M0.reshape(T * C_IN, 3, HALF)
    M = jnp.pad(M1, ((0, 0), (0, 0), (0, LANE - HALF))).reshape(
        T * C_IN, 3 * LANE)
    brow = jnp.pad(
        jnp.tile(jnp.stack([b1, b2, b3])[:, None, :], (1, ATT_DIM, 1))
        .reshape(3, HALF), ((0, 0), (0, LANE - HALF)))
    wab = jnp.pad(Wfc.reshape(2, HALF), ((0, 0), (0, LANE - HALF)))
    wabt = wab.T
    bfc2 = bfc.reshape(1, 1)
    Xf = X.reshape(B, N, T * C_IN)

    return pl.pallas_call(
        _fused_kernel,
        grid=(B, IB),
        in_specs=[
            pl.BlockSpec((1, N, T * C_IN), lambda b, i: (b, 0, 0)),
            pl.BlockSpec((T * C_IN, 3 * LANE), lambda b, i: (0, 0)),
            pl.BlockSpec((3, LANE), lambda b, i: (0, 0)),
            pl.BlockSpec((2, LANE), lambda b, i: (0, 0)),
            pl.BlockSpec((LANE, 2), lambda b, i: (0, 0)),
            pl.BlockSpec((1, 1), lambda b, i: (0, 0)),
            pl.BlockSpec((N, N), lambda b, i: (0, 0)),
        ],
        out_specs=pl.BlockSpec((1, IR, N), lambda b, i: (b, i, 0)),
        out_shape=jax.ShapeDtypeStruct((B, N, N), jnp.float32),
        scratch_shapes=[
            pltpu.VMEM((N, 1), jnp.float32),
            pltpu.VMEM((1, N), jnp.float32),
        ],
    )(Xf, M, brow, wab, wabt, bfc2, A)


# speedup vs baseline: 1.3561x; 1.3561x over previous
import numpy as np

import jax
import jax.numpy as jnp
from jax.experimental import pallas as pl
B, N, T, C_IN = 8, 1000, 12, 64
OUT_CH = 4
ATT_DIM = T - 2
HALF = ATT_DIM * OUT_CH
LANE = 128

_SEL = np.zeros((3, T, ATT_DIM), dtype=np.float32)
for _k in range(3):
    for _t in range(ATT_DIM):
        _SEL[_k, _t + _k, _t] = 1.0


def _fused_kernel(x_ref, m_ref, brow_ref, wab_ref, wabt_ref, bfc_ref,
                  a_ref, out_ref):
    x = x_ref[0]
    y = jnp.dot(x, m_ref[...], preferred_element_type=jnp.float32)
    o1 = y[:, 0:LANE] + brow_ref[0, :][None, :]
    o2 = y[:, LANE:2 * LANE] + brow_ref[1, :][None, :]
    o3 = y[:, 2 * LANE:3 * LANE] + brow_ref[2, :][None, :]
    t = jax.nn.relu(o1 + jax.nn.sigmoid(o2) + o3)
    s1c = jnp.dot(t, wabt_ref[...],
                  preferred_element_type=jnp.float32)[:, 0:1]
    s2r = jax.lax.dot_general(
        wab_ref[...], t, (((1,), (1,)), ((), ())),
        preferred_element_type=jnp.float32)[1:2, :]
    raw = s1c + s2r + bfc_ref[0, 0]
    scores = jnp.where(raw >= 0, raw, 0.01 * raw)
    a = a_ref[...]
    val = jnp.where(a != 0, scores, 0.0)
    m = jnp.max(val, axis=1, keepdims=True)
    e = jnp.exp(val - m)
    s = jnp.sum(e, axis=1, keepdims=True)
    out_ref[0] = e * (1.0 / s) * a


@jax.jit
def kernel(X, A, W1, b1, W2, b2, W3, b3, Wfc, bfc):
    W = jnp.stack([jnp.transpose(w[:, :, 0, :], (2, 1, 0))
                   for w in (W1, W2, W3)])
    sel = jnp.asarray(_SEL)
    M0 = jnp.einsum('kpt,jkco->pcjto', sel, W)
    M1 = M0.reshape(T * C_IN, 3, HALF)
    M = jnp.pad(M1, ((0, 0), (0, 0), (0, LANE - HALF))).reshape(
        T * C_IN, 3 * LANE)
    brow = jnp.pad(
        jnp.tile(jnp.stack([b1, b2, b3])[:, None, :], (1, ATT_DIM, 1))
        .reshape(3, HALF), ((0, 0), (0, LANE - HALF)))
    wab = jnp.pad(Wfc.reshape(2, HALF), ((0, 0), (0, LANE - HALF)))
    wabt = wab.T
    bfc2 = bfc.reshape(1, 1)
    Xf = X.reshape(B, N, T * C_IN)

    return pl.pallas_call(
        _fused_kernel,
        grid=(B,),
        in_specs=[
            pl.BlockSpec((1, N, T * C_IN), lambda b: (b, 0, 0)),
            pl.BlockSpec((T * C_IN, 3 * LANE), lambda b: (0, 0)),
            pl.BlockSpec((3, LANE), lambda b: (0, 0)),
            pl.BlockSpec((2, LANE), lambda b: (0, 0)),
            pl.BlockSpec((LANE, 2), lambda b: (0, 0)),
            pl.BlockSpec((1, 1), lambda b: (0, 0)),
            pl.BlockSpec((N, N), lambda b: (0, 0)),
        ],
        out_specs=pl.BlockSpec((1, N, N), lambda b: (b, 0, 0)),
        out_shape=jax.ShapeDtypeStruct((B, N, N), jnp.float32),
    )(Xf, M, brow, wab, wabt, bfc2, A)
